# baseline (speedup 1.0000x reference)
import functools

import jax
import jax.numpy as jnp
from jax import lax
from jax.experimental import pallas as pl
from jax.experimental.pallas import tpu as pltpu
from jax.experimental.pallas import tpu_sc as plsc

_NC = 2
_NS = 16
_NW = _NC * _NS
_ROWS = 4096
_RW = _ROWS // _NW
_R = 2
_NBLK = _RW // _R
_NSLOT = 6
_DIST = 4
_COLS = 4096
_L = 16


def _sc_body(x_hbm, o_hbm, buf, *sems):
    sin, sout = sems[:_NSLOT], sems[_NSLOT:]
    wid = lax.axis_index("s") * _NC + lax.axis_index("c")
    base = wid * _RW
    lane = lax.broadcasted_iota(jnp.int32, (_L,), 0)
    odd = (lane % 2) == 1

    def start_in(slot, i):
        r0 = base + i * _R
        pltpu.async_copy(x_hbm.at[0, pl.ds(r0, _R)], buf.at[slot, 0], sin[slot])
        pltpu.async_copy(x_hbm.at[1, pl.ds(r0, _R)], buf.at[slot, 1], sin[slot])

    def wait_in(slot, i):
        r0 = base + i * _R
        pltpu.make_async_copy(
            x_hbm.at[0, pl.ds(r0, _R)], buf.at[slot, 0], sin[slot]).wait()
        pltpu.make_async_copy(
            x_hbm.at[1, pl.ds(r0, _R)], buf.at[slot, 1], sin[slot]).wait()

    def start_out(slot, i):
        r0 = base + i * _R
        pltpu.async_copy(buf.at[slot, 0], o_hbm.at[0, pl.ds(r0, _R)], sout[slot])
        pltpu.async_copy(buf.at[slot, 1], o_hbm.at[1, pl.ds(r0, _R)], sout[slot])

    def wait_out(slot, i):
        r0 = base + i * _R
        pltpu.make_async_copy(
            buf.at[slot, 0], o_hbm.at[0, pl.ds(r0, _R)], sout[slot]).wait()
        pltpu.make_async_copy(
            buf.at[slot, 1], o_hbm.at[1, pl.ds(r0, _R)], sout[slot]).wait()

    def compute(slot):
        @plsc.parallel_loop(0, _COLS // _L, 1, unroll=8)
        def col_body(c):
            c16 = c * _L
            for r in range(1, _R, 2):
                v0 = buf[slot, 0, r, pl.ds(c16, _L)]
                v1 = buf[slot, 1, r, pl.ds(c16, _L)]
                buf[slot, 0, r, pl.ds(c16, _L)] = jnp.where(odd, v1, v0)
                buf[slot, 1, r, pl.ds(c16, _L)] = jnp.where(odd, v0, v1)

    def step(i, u, dynamic):
        slot = u % _NSLOT
        pslot = (u + _DIST) % _NSLOT
        if dynamic:
            @pl.when(i >= _NSLOT - _DIST)
            def _():
                wait_out(pslot, i + _DIST - _NSLOT)

            @pl.when(i + _DIST < _NBLK)
            def _():
                start_in(pslot, i + _DIST)
        else:
            if i >= _NSLOT - _DIST:
                wait_out(pslot, i + _DIST - _NSLOT)
            if i + _DIST < _NBLK:
                start_in(pslot, i + _DIST)
        wait_in(slot, i)
        compute(slot)
        start_out(slot, i)

    for p in range(_DIST):
        start_in(p, p)

    _NFULL = _NBLK // _NSLOT * _NSLOT

    def group_body(k, _):
        for u in range(_NSLOT):
            step(k * _NSLOT + u, u, dynamic=True)
        return 0

    lax.fori_loop(0, _NBLK // _NSLOT, group_body, 0)
    for i in range(_NFULL, _NBLK):
        step(i, i, dynamic=False)
    for b in range(max(0, _NBLK - _NSLOT + _DIST), _NBLK):
        wait_out(b % _NSLOT, b)


kernel = functools.partial(
    pl.kernel,
    mesh=plsc.VectorSubcoreMesh(core_axis_name="c", subcore_axis_name="s"),
    out_type=jax.ShapeDtypeStruct((2, _ROWS, _COLS), jnp.float32),
    scratch_types=[pltpu.VMEM((_NSLOT, 2, _R, _COLS), jnp.float32)]
    + [pltpu.SemaphoreType.DMA] * (2 * _NSLOT),
)(_sc_body)
